# baseline (speedup 1.0000x reference)
import jax
import jax.numpy as jnp
from jax import lax
from jax.experimental import pallas as pl
from jax.experimental.pallas import tpu as pltpu
from jax.experimental.pallas import tpu_sc as plsc

W = 121
L = 16
NC, NS = 2, 16
NW = NC * NS

B, K = 16, 17
NBLK = B * K


def _sc_body(f0_hbm, flip_hbm, o0_hbm, flip_v, ibuf, obuf, osem):
  wid = lax.axis_index("s") * NC + lax.axis_index("c")
  pltpu.sync_copy(flip_hbm, flip_v)
  nb = (NBLK - wid + NW - 1) // NW

  def blk_body(j, carry):
    t = wid + NW * j
    b = lax.div(t, K)
    k = lax.rem(t, K)
    fkv = plsc.load_gather(flip_v, [jnp.full((L,), k, dtype=jnp.int32)])
    fk = jnp.max(fkv)
    pltpu.sync_copy(f0_hbm.at[b, fk], ibuf)

    @pl.when(j > 0)
    def _drain():
      pltpu.make_async_copy(obuf, o0_hbm.at[b, k], osem).wait()

    def row_body(y, rcarry):
      for jj in range(8):
        src = 105 - L * jj if jj < 7 else 0
        dst = L * jj if jj < 7 else 105
        obuf[y, 0, pl.ds(dst, L)] = lax.rev(ibuf[y, 0, pl.ds(src, L)], (0,))
      return rcarry

    lax.fori_loop(0, W, row_body, 0)
    pltpu.async_copy(obuf, o0_hbm.at[b, k], osem)
    return carry

  lax.fori_loop(0, nb, blk_body, 0)

  @pl.when(nb > 0)
  def _final_drain():
    pltpu.make_async_copy(obuf, o0_hbm.at[0, 0], osem).wait()


def _tc_body(flip_ref, f1_ref, o1_ref):
  r = lax.broadcasted_iota(jnp.int32, (W, W), 0)
  c = lax.broadcasted_iota(jnp.int32, (W, W), 1)
  perm = jnp.where(r + c == W - 1, jnp.float32(1.0), jnp.float32(0.0))
  for ch in range(2):
    y = jnp.dot(f1_ref[0, 0, :, ch, :], perm,
                precision=lax.Precision.HIGHEST,
                preferred_element_type=jnp.float32)
    o1_ref[0, 0, :, ch, :] = -y if ch == 0 else y


@jax.jit
def kernel(field0, field1, flip_indices):
  f0t = jnp.transpose(field0, (0, 1, 3, 2, 4))
  f1t = jnp.transpose(field1, (0, 1, 3, 2, 4))

  mesh = plsc.VectorSubcoreMesh(core_axis_name="c", subcore_axis_name="s",
                                num_cores=NC, num_subcores=NS)
  sc_fn = pl.kernel(
      _sc_body,
      out_type=[jax.ShapeDtypeStruct((B, K, W, 1, W), jnp.float32)],
      mesh=mesh,
      compiler_params=pltpu.CompilerParams(needs_layout_passes=False),
      scratch_types=[
          pltpu.VMEM((K,), jnp.int32),
          pltpu.VMEM((W, 1, W), jnp.float32),
          pltpu.VMEM((W, 1, W), jnp.float32),
          pltpu.SemaphoreType.DMA,
      ],
  )
  (o0t,) = sc_fn(f0t, flip_indices)

  tc_fn = pl.pallas_call(
      _tc_body,
      grid_spec=pltpu.PrefetchScalarGridSpec(
          num_scalar_prefetch=1,
          grid=(B, K),
          in_specs=[
              pl.BlockSpec((1, 1, W, 2, W),
                           lambda b, k, flip: (b, flip[k], 0, 0, 0)),
          ],
          out_specs=pl.BlockSpec((1, 1, W, 2, W),
                                 lambda b, k, flip: (b, k, 0, 0, 0)),
      ),
      out_shape=jax.ShapeDtypeStruct((B, K, W, 2, W), jnp.float32),
  )
  o1t = tc_fn(flip_indices, f1t)

  return (jnp.transpose(o0t, (0, 1, 3, 2, 4)),
          jnp.transpose(o1t, (0, 1, 3, 2, 4)))


# speedup vs baseline: 1.1102x; 1.1102x over previous
import jax
import jax.numpy as jnp
from jax import lax
from jax.experimental import pallas as pl
from jax.experimental.pallas import tpu as pltpu
from jax.experimental.pallas import tpu_sc as plsc

W = 121
L = 16
NC, NS = 2, 16
NW = NC * NS

B, K = 16, 17
NBLK = B * K


def _sc_body(f0_hbm, flip_hbm, o0_hbm, flip_v, ibuf, obuf, osem):
  wid = lax.axis_index("s") * NC + lax.axis_index("c")
  pltpu.sync_copy(flip_hbm, flip_v)
  nb = (NBLK - wid + NW - 1) // NW

  def blk_body(j, carry):
    t = wid + NW * j
    b = lax.div(t, K)
    k = lax.rem(t, K)
    fkv = plsc.load_gather(flip_v, [jnp.full((L,), k, dtype=jnp.int32)])
    fk = jnp.max(fkv)
    pltpu.sync_copy(f0_hbm.at[b, fk], ibuf)

    @pl.when(j > 0)
    def _drain():
      pltpu.make_async_copy(obuf, o0_hbm.at[b, k], osem).wait()

    def row_body(y, rcarry):
      for jj in range(8):
        src = 105 - L * jj if jj < 7 else 0
        dst = L * jj if jj < 7 else 105
        obuf[y, 0, pl.ds(dst, L)] = lax.rev(ibuf[y, 0, pl.ds(src, L)], (0,))
      return rcarry

    lax.fori_loop(0, W, row_body, 0)
    pltpu.async_copy(obuf, o0_hbm.at[b, k], osem)
    return carry

  lax.fori_loop(0, nb, blk_body, 0)

  @pl.when(nb > 0)
  def _final_drain():
    pltpu.make_async_copy(obuf, o0_hbm.at[0, 0], osem).wait()


def _tc_body(flip_ref, f1_ref, o1_ref):
  r = lax.broadcasted_iota(jnp.int32, (W, W), 0)
  c = lax.broadcasted_iota(jnp.int32, (W, W), 1)
  perm = jnp.where(r + c == W - 1, jnp.float32(1.0), jnp.float32(0.0))
  for ch in range(2):
    y = jnp.dot(f1_ref[0, 0, :, ch, :], perm,
                preferred_element_type=jnp.float32)
    o1_ref[0, 0, :, ch, :] = -y if ch == 0 else y


@jax.jit
def kernel(field0, field1, flip_indices):
  f0t = jnp.transpose(field0, (0, 1, 3, 2, 4))
  f1t = jnp.transpose(field1, (0, 1, 3, 2, 4))

  mesh = plsc.VectorSubcoreMesh(core_axis_name="c", subcore_axis_name="s",
                                num_cores=NC, num_subcores=NS)
  sc_fn = pl.kernel(
      _sc_body,
      out_type=[jax.ShapeDtypeStruct((B, K, W, 1, W), jnp.float32)],
      mesh=mesh,
      compiler_params=pltpu.CompilerParams(needs_layout_passes=False),
      scratch_types=[
          pltpu.VMEM((K,), jnp.int32),
          pltpu.VMEM((W, 1, W), jnp.float32),
          pltpu.VMEM((W, 1, W), jnp.float32),
          pltpu.SemaphoreType.DMA,
      ],
  )
  (o0t,) = sc_fn(f0t, flip_indices)

  tc_fn = pl.pallas_call(
      _tc_body,
      grid_spec=pltpu.PrefetchScalarGridSpec(
          num_scalar_prefetch=1,
          grid=(B, K),
          in_specs=[
              pl.BlockSpec((1, 1, W, 2, W),
                           lambda b, k, flip: (b, flip[k], 0, 0, 0)),
          ],
          out_specs=pl.BlockSpec((1, 1, W, 2, W),
                                 lambda b, k, flip: (b, k, 0, 0, 0)),
      ),
      out_shape=jax.ShapeDtypeStruct((B, K, W, 2, W), jnp.float32),
  )
  o1t = tc_fn(flip_indices, f1t)

  return (jnp.transpose(o0t, (0, 1, 3, 2, 4)),
          jnp.transpose(o1t, (0, 1, 3, 2, 4)))


# speedup vs baseline: 2.8202x; 2.5402x over previous
import jax
import jax.numpy as jnp
from jax import lax
from jax.experimental import pallas as pl
from jax.experimental.pallas import tpu as pltpu
from jax.experimental.pallas import tpu_sc as plsc

W = 121
L = 16
NC, NS = 2, 16
NW = NC * NS

B, K = 16, 17
NBLK = B * K


def _body(f0_hbm, f1_hbm, flip_hbm, o0_hbm, o1_hbm,
          flip_v, i0, o0, i1, o1, osem):
  wid = lax.axis_index("s") * NC + lax.axis_index("c")
  pltpu.sync_copy(flip_hbm, flip_v)

  def do_field(in_hbm, out_hbm, ibuf, obuf, c_dim, signed):
    nb = (NBLK - wid + NW - 1) // NW

    def blk_body(j, carry):
      t = wid + NW * j
      b = lax.div(t, K)
      k = lax.rem(t, K)
      fkv = plsc.load_gather(flip_v, [jnp.full((L,), k, dtype=jnp.int32)])
      fk = jnp.max(fkv)
      pltpu.sync_copy(in_hbm.at[b, fk], ibuf)

      @pl.when(j > 0)
      def _drain():
        pltpu.make_async_copy(obuf, out_hbm.at[b, k], osem).wait()

      def row_body(y, rcarry):
        for c in range(c_dim):
          neg = signed and c == 0
          for jj in range(8):
            src = 105 - L * jj if jj < 7 else 0
            dst = L * jj if jj < 7 else 105
            v = lax.rev(ibuf[y, c, pl.ds(src, L)], (0,))
            if neg:
              v = -v
            obuf[y, c, pl.ds(dst, L)] = v
        return rcarry

      lax.fori_loop(0, W, row_body, 0)
      pltpu.async_copy(obuf, out_hbm.at[b, k], osem)
      return carry

    lax.fori_loop(0, nb, blk_body, 0)
    @pl.when(nb > 0)
    def _final_drain():
      pltpu.make_async_copy(obuf, out_hbm.at[0, 0], osem).wait()

  do_field(f0_hbm, o0_hbm, i0, o0, 1, False)
  do_field(f1_hbm, o1_hbm, i1, o1, 2, True)


@jax.jit
def kernel(field0, field1, flip_indices):
  mesh = plsc.VectorSubcoreMesh(core_axis_name="c", subcore_axis_name="s",
                                num_cores=NC, num_subcores=NS)
  fn = pl.kernel(
      _body,
      out_type=[
          jax.ShapeDtypeStruct((B, K, W, 1, W), jnp.float32),
          jax.ShapeDtypeStruct((B, K, W, 2, W), jnp.float32),
      ],
      mesh=mesh,
      compiler_params=pltpu.CompilerParams(needs_layout_passes=False),
      scratch_types=[
          pltpu.VMEM((K,), jnp.int32),
          pltpu.VMEM((W, 1, W), jnp.float32),
          pltpu.VMEM((W, 1, W), jnp.float32),
          pltpu.VMEM((W, 2, W), jnp.float32),
          pltpu.VMEM((W, 2, W), jnp.float32),
          pltpu.SemaphoreType.DMA,
      ],
  )
  f0t = jnp.transpose(field0, (0, 1, 3, 2, 4))
  f1t = jnp.transpose(field1, (0, 1, 3, 2, 4))
  o0t, o1t = fn(f0t, f1t, flip_indices)
  return (jnp.transpose(o0t, (0, 1, 3, 2, 4)),
          jnp.transpose(o1t, (0, 1, 3, 2, 4)))
